# baseline (speedup 1.0000x reference)
import functools

import numpy as np
import jax
import jax.numpy as jnp
from jax import lax
from jax.experimental import pallas as pl
from jax.experimental.pallas import tpu as pltpu
from jax.experimental.pallas import tpu_sc as plsc

_BOX = 100
_B, _C, _H, _W = 64, 3, 224, 224
_NLAB = 1000
_BATCH_PROB = 0.1


def _static_rng():
    rs = np.random.RandomState(42)
    perm = rs.permutation(_B)
    keep = rs.rand(_B) > _BATCH_PROB
    xs = rs.randint(0, _H - _BOX + 1, size=_B)
    ys = rs.randint(0, _W - _BOX + 1, size=_B)
    return perm, keep, xs, ys


_PERM, _KEEP, _XS, _YS = _static_rng()
_KEEP_IDX = np.nonzero(_KEEP)[0]
_K = int(len(_KEEP_IDX))
_LAM = 1.0 - (_BOX * _BOX) / float(_H * _W)

_EYE = np.eye(_B, dtype=np.float32)
_WLAB = (_LAM * _EYE[_KEEP_IDX] + (1.0 - _LAM) * _EYE[_PERM[_KEEP_IDX]])

_NUM_TILES = 32
_L = 16
_HH = 112


def _merge_rows(pv, bv, h0, wa, rlo, rhi, y, k_lo, k_hi, m_l, m_r):
    lo_l = pl.multiple_of(k_lo * _L, _L)
    lo_r = pl.multiple_of(k_hi * _L, _L)
    mids = []
    for k in range(1, 7):
        lo = jnp.minimum((k_lo + k) * _L, (k_hi - 1) * _L)
        mids.append(pl.multiple_of(lo, _L))

    @plsc.parallel_loop(rlo, rhi, unroll=8)
    def row(r):
        rb = r - wa
        rp = r - h0
        s = bv[rb, pl.ds(lo_l, _L)]
        cur = pv[rp, pl.ds(lo_l, _L)]
        pv[rp, pl.ds(lo_l, _L)] = jnp.where(m_l, s, cur)
        s = bv[rb, pl.ds(lo_r, _L)]
        cur = pv[rp, pl.ds(lo_r, _L)]
        pv[rp, pl.ds(lo_r, _L)] = jnp.where(m_r, s, cur)
        for lo in mids:
            pv[rp, pl.ds(lo, _L)] = bv[rb, pl.ds(lo, _L)]


def _sc_images_body(images_hbm, out_hbm, pv0, pv1, bv0, smem,
                    sem_p0, sem_p1, sem_b0, sem_o0, sem_o1):
    wid = lax.axis_index("s") * 2 + lax.axis_index("c")

    for t in range(_NUM_TILES):
        my = [i for i in range(_K) if i % _NUM_TILES == t]
        if not my:
            continue

        @pl.when(wid == t)
        def _params(my=my):
            for j, i in enumerate(my):
                b = int(_KEEP_IDX[i])
                smem[5 * j + 0] = b
                smem[5 * j + 1] = int(_PERM[b])
                smem[5 * j + 2] = int(_XS[b])
                smem[5 * j + 3] = int(_YS[b])
                smem[5 * j + 4] = i
            smem[15] = len(my)

    n_pairs = smem[15] * _C
    col0 = lax.iota(jnp.int32, _L)

    def pair(p, carry):
        j = p // _C
        c = p - j * _C
        b = smem[5 * j + 0]
        pb = smem[5 * j + 1]
        x = smem[5 * j + 2]
        y = smem[5 * j + 3]
        oi = smem[5 * j + 4]

        k_lo = y // _L
        k_hi = (y + _BOX - 1) // _L
        col_l = k_lo * _L + col0
        m_l = (col_l >= y) & (col_l < y + _BOX)
        col_r = k_hi * _L + col0
        m_r = (col_r >= y) & (col_r < y + _BOX)

        rlo0 = x
        rhi0 = jnp.minimum(x + _BOX, _HH)
        has0 = x < _HH
        rlo1 = jnp.maximum(x, _HH)
        rhi1 = x + _BOX
        has1 = x + _BOX > _HH
        wa = pl.multiple_of(jnp.minimum((x // 8) * 8, _H - _HH), 8)

        pltpu.async_copy(images_hbm.at[pb, c, pl.ds(wa, _HH)], bv0, sem_b0)

        @pl.when(p > 0)
        def _drain0():
            pltpu.make_async_copy(
                pv0, out_hbm.at[0, 0, pl.ds(0, _HH)], sem_o0
            ).wait()

        cp0 = pltpu.async_copy(images_hbm.at[b, c, pl.ds(0, _HH)], pv0, sem_p0)

        @pl.when(p > 0)
        def _drain1():
            pltpu.make_async_copy(
                pv1, out_hbm.at[0, 0, pl.ds(_HH, _HH)], sem_o1
            ).wait()

        cp1 = pltpu.async_copy(
            images_hbm.at[b, c, pl.ds(_HH, _HH)], pv1, sem_p1
        )

        cp0.wait()
        pltpu.make_async_copy(
            images_hbm.at[0, 0, pl.ds(0, _HH)], bv0, sem_b0
        ).wait()

        @pl.when(has0)
        def _merge0():
            _merge_rows(pv0, bv0, 0, wa, rlo0, rhi0, y, k_lo, k_hi, m_l, m_r)

        pltpu.async_copy(pv0, out_hbm.at[oi, c, pl.ds(0, _HH)], sem_o0)

        cp1.wait()

        @pl.when(has1)
        def _merge1():
            _merge_rows(pv1, bv0, _HH, wa, rlo1, rhi1, y, k_lo, k_hi,
                        m_l, m_r)

        pltpu.async_copy(pv1, out_hbm.at[oi, c, pl.ds(_HH, _HH)], sem_o1)
        return carry

    lax.fori_loop(0, n_pairs, pair, 0)

    pltpu.make_async_copy(pv0, out_hbm.at[0, 0, pl.ds(0, _HH)], sem_o0).wait()
    pltpu.make_async_copy(
        pv1, out_hbm.at[0, 0, pl.ds(_HH, _HH)], sem_o1
    ).wait()


_sc_images = pl.kernel(
    _sc_images_body,
    out_type=jax.ShapeDtypeStruct((_K, _C, _H, _W), jnp.float32),
    mesh=plsc.VectorSubcoreMesh(core_axis_name="c", subcore_axis_name="s"),
    scratch_types=[
        pltpu.VMEM((_HH, _W), jnp.float32),
        pltpu.VMEM((_HH, _W), jnp.float32),
        pltpu.VMEM((_HH, _W), jnp.float32),
        pltpu.SMEM((16,), jnp.int32),
        pltpu.SemaphoreType.DMA,
        pltpu.SemaphoreType.DMA,
        pltpu.SemaphoreType.DMA,
        pltpu.SemaphoreType.DMA,
        pltpu.SemaphoreType.DMA,
    ],
)


def _tc_labels_body(w_ref, l_ref, o_ref):
    o_ref[...] = jnp.dot(
        w_ref[...], l_ref[...], preferred_element_type=jnp.float32
    )


def _tc_labels(labels):
    return pl.pallas_call(
        _tc_labels_body,
        out_shape=jax.ShapeDtypeStruct((_K, _NLAB), jnp.float32),
    )(jnp.asarray(_WLAB), labels)


@jax.jit
def kernel(images, labels):
    mixed = _sc_images(images)
    mixed_labels = _tc_labels(labels)
    return mixed, mixed_labels


# speedup vs baseline: 1.0746x; 1.0746x over previous
import functools

import numpy as np
import jax
import jax.numpy as jnp
from jax import lax
from jax.experimental import pallas as pl
from jax.experimental.pallas import tpu as pltpu
from jax.experimental.pallas import tpu_sc as plsc

_BOX = 100
_B, _C, _H, _W = 64, 3, 224, 224
_NLAB = 1000
_BATCH_PROB = 0.1


def _static_rng():
    rs = np.random.RandomState(42)
    perm = rs.permutation(_B)
    keep = rs.rand(_B) > _BATCH_PROB
    xs = rs.randint(0, _H - _BOX + 1, size=_B)
    ys = rs.randint(0, _W - _BOX + 1, size=_B)
    return perm, keep, xs, ys


_PERM, _KEEP, _XS, _YS = _static_rng()
_KEEP_IDX = np.nonzero(_KEEP)[0]
_K = int(len(_KEEP_IDX))
_LAM = 1.0 - (_BOX * _BOX) / float(_H * _W)

_EYE = np.eye(_B, dtype=np.float32)
_WLAB = (_LAM * _EYE[_KEEP_IDX] + (1.0 - _LAM) * _EYE[_PERM[_KEEP_IDX]])

_NUM_TILES = 32
_L = 16
_HH = 112


def _merge_rows(pv, bv, h0, wa, rlo, rhi, y, k_lo, k_hi, m_l, m_r):
    lo_l = pl.multiple_of(k_lo * _L, _L)
    lo_r = pl.multiple_of(k_hi * _L, _L)
    mids = []
    for k in range(1, 7):
        lo = jnp.minimum((k_lo + k) * _L, (k_hi - 1) * _L)
        mids.append(pl.multiple_of(lo, _L))

    @plsc.parallel_loop(rlo, rhi, unroll=4)
    def row(r):
        rb = r - wa
        rp = r - h0
        s = bv[rb, pl.ds(lo_l, _L)]
        cur = pv[rp, pl.ds(lo_l, _L)]
        pv[rp, pl.ds(lo_l, _L)] = jnp.where(m_l, s, cur)
        s = bv[rb, pl.ds(lo_r, _L)]
        cur = pv[rp, pl.ds(lo_r, _L)]
        pv[rp, pl.ds(lo_r, _L)] = jnp.where(m_r, s, cur)
        for lo in mids:
            pv[rp, pl.ds(lo, _L)] = bv[rb, pl.ds(lo, _L)]


def _sc_images_body(images_hbm, out_hbm, pv0, pv1, bv0, smem,
                    sem_p0, sem_p1, sem_b0, sem_o0, sem_o1):
    wid = lax.axis_index("s") * 2 + lax.axis_index("c")

    for t in range(_NUM_TILES):
        my = [i for i in range(_K) if i % _NUM_TILES == t]
        if not my:
            continue

        @pl.when(wid == t)
        def _params(my=my):
            for j, i in enumerate(my):
                b = int(_KEEP_IDX[i])
                smem[5 * j + 0] = b
                smem[5 * j + 1] = int(_PERM[b])
                smem[5 * j + 2] = int(_XS[b])
                smem[5 * j + 3] = int(_YS[b])
                smem[5 * j + 4] = i
            smem[15] = len(my)

    n_pairs = smem[15] * _C
    col0 = lax.iota(jnp.int32, _L)

    def pair(p, carry):
        j = p // _C
        c = p - j * _C
        b = smem[5 * j + 0]
        pb = smem[5 * j + 1]
        x = smem[5 * j + 2]
        y = smem[5 * j + 3]
        oi = smem[5 * j + 4]

        k_lo = y // _L
        k_hi = (y + _BOX - 1) // _L
        col_l = k_lo * _L + col0
        m_l = (col_l >= y) & (col_l < y + _BOX)
        col_r = k_hi * _L + col0
        m_r = (col_r >= y) & (col_r < y + _BOX)

        rlo0 = x
        rhi0 = jnp.minimum(x + _BOX, _HH)
        has0 = x < _HH
        rlo1 = jnp.maximum(x, _HH)
        rhi1 = x + _BOX
        has1 = x + _BOX > _HH
        wa = pl.multiple_of(jnp.minimum((x // 8) * 8, _H - _HH), 8)

        pltpu.async_copy(images_hbm.at[pb, c, pl.ds(wa, _HH)], bv0, sem_b0)

        @pl.when(p > 0)
        def _drain0():
            pltpu.make_async_copy(
                pv0, out_hbm.at[0, 0, pl.ds(0, _HH)], sem_o0
            ).wait()

        cp0 = pltpu.async_copy(images_hbm.at[b, c, pl.ds(0, _HH)], pv0, sem_p0)

        @pl.when(p > 0)
        def _drain1():
            pltpu.make_async_copy(
                pv1, out_hbm.at[0, 0, pl.ds(_HH, _HH)], sem_o1
            ).wait()

        cp1 = pltpu.async_copy(
            images_hbm.at[b, c, pl.ds(_HH, _HH)], pv1, sem_p1
        )

        cp0.wait()
        pltpu.make_async_copy(
            images_hbm.at[0, 0, pl.ds(0, _HH)], bv0, sem_b0
        ).wait()

        @pl.when(has0)
        def _merge0():
            _merge_rows(pv0, bv0, 0, wa, rlo0, rhi0, y, k_lo, k_hi, m_l, m_r)

        pltpu.async_copy(pv0, out_hbm.at[oi, c, pl.ds(0, _HH)], sem_o0)

        cp1.wait()

        @pl.when(has1)
        def _merge1():
            _merge_rows(pv1, bv0, _HH, wa, rlo1, rhi1, y, k_lo, k_hi,
                        m_l, m_r)

        pltpu.async_copy(pv1, out_hbm.at[oi, c, pl.ds(_HH, _HH)], sem_o1)
        return carry

    lax.fori_loop(0, n_pairs, pair, 0)

    pltpu.make_async_copy(pv0, out_hbm.at[0, 0, pl.ds(0, _HH)], sem_o0).wait()
    pltpu.make_async_copy(
        pv1, out_hbm.at[0, 0, pl.ds(_HH, _HH)], sem_o1
    ).wait()


_sc_images = pl.kernel(
    _sc_images_body,
    out_type=jax.ShapeDtypeStruct((_K, _C, _H, _W), jnp.float32),
    mesh=plsc.VectorSubcoreMesh(core_axis_name="c", subcore_axis_name="s"),
    scratch_types=[
        pltpu.VMEM((_HH, _W), jnp.float32),
        pltpu.VMEM((_HH, _W), jnp.float32),
        pltpu.VMEM((_HH, _W), jnp.float32),
        pltpu.SMEM((16,), jnp.int32),
        pltpu.SemaphoreType.DMA,
        pltpu.SemaphoreType.DMA,
        pltpu.SemaphoreType.DMA,
        pltpu.SemaphoreType.DMA,
        pltpu.SemaphoreType.DMA,
    ],
)


def _tc_labels_body(w_ref, l_ref, o_ref):
    o_ref[...] = jnp.dot(
        w_ref[...], l_ref[...], preferred_element_type=jnp.float32
    )


def _tc_labels(labels):
    return pl.pallas_call(
        _tc_labels_body,
        out_shape=jax.ShapeDtypeStruct((_K, _NLAB), jnp.float32),
    )(jnp.asarray(_WLAB), labels)


@jax.jit
def kernel(images, labels):
    mixed = _sc_images(images)
    mixed_labels = _tc_labels(labels)
    return mixed, mixed_labels
